# baseline (speedup 1.0000x reference)
import numpy as np
import jax
import jax.numpy as jnp
from jax import lax
from jax.experimental import pallas as pl
from jax.experimental.pallas import tpu as pltpu

B = 2; T = 2048; V = 8; P = 64; NPATCH = T // P; C = 768; H = 12; DH = C // H
LAYERS = 2; NEXP = 8; TOPK = 2; DFF = 3072; K = 10; L = NPATCH + 1
FREQ_P = P // 2 + 1; FREQ_S = T // 2 + 1
NSEQ = B * V
LP = 40
NTOK = NSEQ * LP
NREAL = NSEQ * L
TT = 128
MAXT = (NREAL * TOPK + NEXP * (TT - 1)) // TT
SLOTS = MAXT * TT


def _dft_mats(n, nfreq):
    ns = np.arange(n)[:, None]
    ks = np.arange(nfreq)[None, :]
    ang = 2.0 * np.pi * ns * ks / n
    return np.cos(ang).astype(np.float32), np.sin(ang).astype(np.float32)

_DPC, _DPS = _dft_mats(P, FREQ_P)
_DSC, _DSS = _dft_mats(T, FREQ_S)
_SU128 = np.triu(np.ones((128, 128), np.float32), 1)
_SL640 = np.tril(np.ones((NTOK, NTOK), np.float32), -1)


def _ln_in(x, s, b):
    m = jnp.mean(x, axis=-1, keepdims=True)
    v = jnp.mean((x - m) * (x - m), axis=-1, keepdims=True)
    return (x - m) * lax.rsqrt(v + 1e-6) * s + b


def _dot(a, b):
    return jnp.dot(a, b, preferred_element_type=jnp.float32)


def _dot_t(a, b):
    return lax.dot_general(a, b, (((1,), (1,)), ((), ())),
                           preferred_element_type=jnp.float32)


def _dot_tl(a, b):
    return lax.dot_general(a, b, (((0,), (0,)), ((), ())),
                           preferred_element_type=jnp.float32)


def _tokenize_kernel(patches_ref, x_ref, dpc_ref, dps_ref, wp_ref,
                     dsc_ref, dss_ref, ws_ref, pos_ref, h_ref):
    pr = patches_ref[...]
    re = _dot(pr, dpc_ref[...])
    im = _dot(pr, dps_ref[...])
    pf = jnp.sqrt(re * re + im * im)
    tok = _dot(pf, wp_ref[...])
    xr = x_ref[...]
    sre = _dot(xr, dsc_ref[...])
    sim_ = _dot(xr, dss_ref[...])
    sf = jnp.sqrt(sre * sre + sim_ * sim_)
    cls = _dot(sf, ws_ref[...])
    pos = pos_ref[...]
    zero = jnp.zeros((LP - L, C), jnp.float32)
    for s in range(NSEQ):
        h_ref[s * LP:s * LP + 1, :] = cls[s:s + 1, :] + pos[0:1, :]
        h_ref[s * LP + 1:s * LP + L, :] = tok[s * NPATCH:(s + 1) * NPATCH, :] + pos[1:L, :]
        h_ref[s * LP + L:(s + 1) * LP, :] = zero


def _attn_kernel(h_ref, s_ref, b_ref, wqkv_ref, wo_ref, out_ref):
    hs = h_ref[0]
    y = _ln_in(hs, s_ref[...], b_ref[...])
    qkv = _dot(y, wqkv_ref[...])
    col = lax.broadcasted_iota(jnp.int32, (LP, LP), 1)
    mask = jnp.where(col < L, 0.0, -1e9).astype(jnp.float32)
    pieces = []
    for h in range(H):
        q = qkv[:, h * DH:(h + 1) * DH]
        k = qkv[:, C + h * DH:C + (h + 1) * DH]
        v = qkv[:, 2 * C + h * DH:2 * C + (h + 1) * DH]
        sc = _dot_t(q, k) * (1.0 / np.sqrt(float(DH))) + mask
        m = jnp.max(sc, axis=-1, keepdims=True)
        e = jnp.exp(sc - m)
        p = e / jnp.sum(e, axis=-1, keepdims=True)
        pieces.append(_dot(p, v))
    o = jnp.concatenate(pieces, axis=1)
    out_ref[0] = hs + _dot(o, wo_ref[...])


def _router_kernel(h_ref, s_ref, b_ref, wr_ref, su_ref, sl_ref,
                   y2_ref, pg_ref, ps_ref, te_ref, cmb_ref):
    h = h_ref[...]
    y2 = _ln_in(h, s_ref[...], b_ref[...])
    y2_ref[...] = y2
    logits = _dot(y2, wr_ref[...])
    col = lax.broadcasted_iota(jnp.int32, (NTOK, 128), 1)
    logits = jnp.where(col < NEXP, logits, -1e30)
    mx = jnp.max(logits, axis=-1, keepdims=True)
    ex = jnp.exp(logits - mx)
    probs = ex / jnp.sum(ex, axis=-1, keepdims=True)
    su = su_ref[...]
    m1 = jnp.max(probs, axis=-1, keepdims=True)
    eq1 = (probs == m1).astype(jnp.float32)
    first = eq1 * (1.0 - jnp.minimum(_dot(eq1, su), 1.0))
    probs2 = probs - first * 2.0
    m2 = jnp.max(probs2, axis=-1, keepdims=True)
    eq2 = (probs2 == m2).astype(jnp.float32)
    second = eq2 * (1.0 - jnp.minimum(_dot(eq2, su), 1.0))
    denom = m1 + m2
    w1 = m1 / denom
    w2 = m2 / denom
    row = lax.broadcasted_iota(jnp.int32, (NTOK, 128), 0)
    validf = ((row % LP) < L).astype(jnp.float32)
    first = first * validf
    second = second * validf
    sl = sl_ref[...]
    cnt1 = jnp.sum(first, axis=0, keepdims=True)
    cnt2 = jnp.sum(second, axis=0, keepdims=True)
    r0 = _dot(sl, first)
    r1 = cnt1 + _dot(sl, second)
    cnt = cnt1 + cnt2
    pc = jnp.floor((cnt + float(TT - 1)) * (1.0 / TT)) * float(TT)
    base = _dot(pc, su)
    pos0 = jnp.sum(first * (base + r0), axis=-1, keepdims=True)
    pos1 = jnp.sum(second * (base + r1), axis=-1, keepdims=True)
    rowv1 = (lax.broadcasted_iota(jnp.int32, (NTOK, 1), 0) % LP) < L
    big = float(SLOTS + 7)
    pos0 = jnp.where(rowv1, pos0, big)
    pos1 = jnp.where(rowv1, pos1, big)
    gi = lax.broadcasted_iota(jnp.int32, (NTOK, SLOTS), 1)
    hit0 = (gi == pos0.astype(jnp.int32)).astype(jnp.float32)
    hit1 = (gi == pos1.astype(jnp.int32)).astype(jnp.float32)
    pg_ref[...] = hit0 + hit1
    ps_ref[...] = hit0 * w1 + hit1 * w2
    ebase = base * (1.0 / TT)
    etiles = pc * (1.0 / TT)
    eye = (lax.broadcasted_iota(jnp.int32, (128, 128), 0) ==
           lax.broadcasted_iota(jnp.int32, (128, 128), 1)).astype(jnp.float32)
    bcolT = _dot_t(eye, ebase)
    ncolT = _dot_t(eye, etiles)
    ti = lax.broadcasted_iota(jnp.int32, (1, MAXT), 1).astype(jnp.float32)
    ind = jnp.logical_and(ti >= bcolT, ti < bcolT + ncolT).astype(jnp.float32)
    evec = lax.broadcasted_iota(jnp.int32, (1, 128), 1).astype(jnp.float32)
    te = _dot(evec, ind)
    te_ref[...] = jnp.broadcast_to(te, (8, MAXT)).astype(jnp.int32)
    cmb_ref[...] = first * w1 + second * w2


def _moe_grouped_kernel(te_ref, pg_ref, ps_ref, y2_ref, h_ref,
                        we1_ref, we2_ref, out_ref):
    t = pl.program_id(0)

    @pl.when(t == 0)
    def _():
        out_ref[...] = h_ref[...]

    x = _dot_tl(pg_ref[...], y2_ref[...])
    a = jax.nn.gelu(_dot(x.astype(jnp.bfloat16), we1_ref[0]))
    o = _dot(a.astype(jnp.bfloat16), we2_ref[0])
    out_ref[...] += _dot(ps_ref[...], o)


def _moe_dense_kernel(y2_ref, cmb_ref, h_ref, we1_ref, we2_ref, out_ref):
    e = pl.program_id(0)

    @pl.when(e == 0)
    def _():
        out_ref[...] = h_ref[...]

    a = jax.nn.gelu(_dot(y2_ref[...], we1_ref[0]))
    o = _dot(a.astype(jnp.bfloat16), we2_ref[0])
    col = lax.broadcasted_iota(jnp.int32, (NTOK, 128), 1)
    w = jnp.sum(jnp.where(col == e, cmb_ref[...], 0.0), axis=-1, keepdims=True)
    out_ref[...] += o * w


def _head_kernel(clst_ref, wcls_ref, bcls_ref, cat_ref, m_ref, out_ref):
    proj = _dot(clst_ref[...], wcls_ref[...]) + bcls_ref[...]
    pn = proj / (jnp.sqrt(jnp.sum(proj * proj, axis=-1, keepdims=True)) + 1e-8)
    ct = cat_ref[...]
    cn = ct / (jnp.sqrt(jnp.sum(ct * ct, axis=-1, keepdims=True)) + 1e-8)
    sim = _dot_t(pn, cn)
    out_ref[...] = _dot(m_ref[...], sim)


@jax.jit
def _run(x_enc, W_patch, W_seq, pos_emb, ln1_s, ln1_b, Wqkv, Wo,
         ln2_s, ln2_b, Wr, We1, We2, Wcls, bcls, cat_tok):
    f32 = jnp.float32
    xt = jnp.transpose(x_enc, (0, 2, 1)).reshape(NSEQ, T)
    patches = xt.reshape(NSEQ * NPATCH, P)
    pos_p = jnp.zeros((LP, C), f32).at[:L].set(pos_emb)

    h = pl.pallas_call(
        _tokenize_kernel,
        out_shape=jax.ShapeDtypeStruct((NTOK, C), f32),
    )(patches, xt, jnp.asarray(_DPC), jnp.asarray(_DPS), W_patch,
      jnp.asarray(_DSC), jnp.asarray(_DSS), W_seq, pos_p)

    su = jnp.asarray(_SU128)
    sl = jnp.asarray(_SL640)
    wr_p = jnp.zeros((C, 128), f32)
    full = lambda shp: pl.BlockSpec(shp, lambda *_: (0,) * len(shp))

    for l in range(LAYERS):
        h3 = h.reshape(NSEQ, LP, C)
        h3 = pl.pallas_call(
            _attn_kernel,
            grid=(NSEQ,),
            in_specs=[
                pl.BlockSpec((1, LP, C), lambda s: (s, 0, 0)),
                full((1, C)), full((1, C)),
                full((C, 3 * C)), full((C, C)),
            ],
            out_specs=pl.BlockSpec((1, LP, C), lambda s: (s, 0, 0)),
            out_shape=jax.ShapeDtypeStruct((NSEQ, LP, C), f32),
        )(h3, ln1_s[l][None], ln1_b[l][None], Wqkv[l], Wo[l])
        h = h3.reshape(NTOK, C)

        y2, pg, ps, te, cmb = pl.pallas_call(
            _router_kernel,
            out_shape=[jax.ShapeDtypeStruct((NTOK, C), f32),
                       jax.ShapeDtypeStruct((NTOK, SLOTS), f32),
                       jax.ShapeDtypeStruct((NTOK, SLOTS), f32),
                       jax.ShapeDtypeStruct((8, MAXT), jnp.int32),
                       jax.ShapeDtypeStruct((NTOK, 128), f32)],
        )(h, ln2_s[l][None], ln2_b[l][None], wr_p.at[:, :NEXP].set(Wr[l]),
          su, sl)

        bf16 = jnp.bfloat16
        h = pl.pallas_call(
            _moe_dense_kernel,
            grid=(NEXP,),
            in_specs=[
                full((NTOK, C)), full((NTOK, 128)), full((NTOK, C)),
                pl.BlockSpec((1, C, DFF), lambda e: (e, 0, 0)),
                pl.BlockSpec((1, DFF, C), lambda e: (e, 0, 0)),
            ],
            out_specs=full((NTOK, C)),
            out_shape=jax.ShapeDtypeStruct((NTOK, C), f32),
        )(y2.astype(bf16), cmb, h, We1[l].astype(bf16), We2[l].astype(bf16))

    clst = h.reshape(NSEQ, LP, C)[:, 0, :]
    cat_p = jnp.zeros((16, C), f32).at[:K].set(cat_tok)
    mmat = np.zeros((8, 16), np.float32)
    for b in range(B):
        mmat[b, b * V:(b + 1) * V] = 1.0 / V
    out = pl.pallas_call(
        _head_kernel,
        out_shape=jax.ShapeDtypeStruct((8, 16), f32),
    )(clst, Wcls, bcls[None], cat_p, jnp.asarray(mmat))
    return out[:B, :K]


def kernel(x_enc, x_mark_enc, W_patch, W_seq, pos_emb, ln1_s, ln1_b, Wqkv, Wo,
           ln2_s, ln2_b, Wr, We1, We2, Wcls, bcls, cat_tok):
    return _run(x_enc, W_patch, W_seq, pos_emb, ln1_s, ln1_b, Wqkv, Wo,
                ln2_s, ln2_b, Wr, We1, We2, Wcls, bcls, cat_tok)


# speedup vs baseline: 1.0788x; 1.0788x over previous
import numpy as np
import jax
import jax.numpy as jnp
from jax import lax
from jax.experimental import pallas as pl
from jax.experimental.pallas import tpu as pltpu

B = 2; T = 2048; V = 8; P = 64; NPATCH = T // P; C = 768; H = 12; DH = C // H
LAYERS = 2; NEXP = 8; TOPK = 2; DFF = 3072; K = 10; L = NPATCH + 1
FREQ_P = P // 2 + 1; FREQ_S = T // 2 + 1
NSEQ = B * V
LP = 40
NTOK = NSEQ * LP
NREAL = NSEQ * L
TT = 128
MAXT = (NREAL * TOPK + NEXP * (TT - 1)) // TT
SLOTS = MAXT * TT


def _dft_mats(n, nfreq):
    ns = np.arange(n)[:, None]
    ks = np.arange(nfreq)[None, :]
    ang = 2.0 * np.pi * ns * ks / n
    return np.cos(ang).astype(np.float32), np.sin(ang).astype(np.float32)

_DPC, _DPS = _dft_mats(P, FREQ_P)
_DSC, _DSS = _dft_mats(T, FREQ_S)
_SU128 = np.triu(np.ones((128, 128), np.float32), 1)
_SL640 = np.tril(np.ones((NTOK, NTOK), np.float32), -1)


def _ln_in(x, s, b):
    m = jnp.mean(x, axis=-1, keepdims=True)
    v = jnp.mean((x - m) * (x - m), axis=-1, keepdims=True)
    return (x - m) * lax.rsqrt(v + 1e-6) * s + b


def _dot(a, b):
    return jnp.dot(a, b, preferred_element_type=jnp.float32)


def _dot_t(a, b):
    return lax.dot_general(a, b, (((1,), (1,)), ((), ())),
                           preferred_element_type=jnp.float32)


def _dot_tl(a, b):
    return lax.dot_general(a, b, (((0,), (0,)), ((), ())),
                           preferred_element_type=jnp.float32)


def _tokenize_kernel(patches_ref, x_ref, dpc_ref, dps_ref, wp_ref,
                     dsc_ref, dss_ref, ws_ref, pos_ref, h_ref):
    pr = patches_ref[...]
    re = _dot(pr, dpc_ref[...])
    im = _dot(pr, dps_ref[...])
    pf = jnp.sqrt(re * re + im * im)
    tok = _dot(pf, wp_ref[...])
    xr = x_ref[...]
    sre = _dot(xr, dsc_ref[...])
    sim_ = _dot(xr, dss_ref[...])
    sf = jnp.sqrt(sre * sre + sim_ * sim_)
    cls = _dot(sf, ws_ref[...])
    pos = pos_ref[...]
    zero = jnp.zeros((LP - L, C), jnp.float32)
    for s in range(NSEQ):
        h_ref[s * LP:s * LP + 1, :] = cls[s:s + 1, :] + pos[0:1, :]
        h_ref[s * LP + 1:s * LP + L, :] = tok[s * NPATCH:(s + 1) * NPATCH, :] + pos[1:L, :]
        h_ref[s * LP + L:(s + 1) * LP, :] = zero


def _attn_kernel(h_ref, s_ref, b_ref, wqkv_ref, wo_ref, out_ref):
    hs = h_ref[0]
    y = _ln_in(hs, s_ref[...], b_ref[...])
    qkv = _dot(y, wqkv_ref[...])
    col = lax.broadcasted_iota(jnp.int32, (LP, LP), 1)
    mask = jnp.where(col < L, 0.0, -1e9).astype(jnp.float32)
    pieces = []
    for h in range(H):
        q = qkv[:, h * DH:(h + 1) * DH]
        k = qkv[:, C + h * DH:C + (h + 1) * DH]
        v = qkv[:, 2 * C + h * DH:2 * C + (h + 1) * DH]
        sc = _dot_t(q, k) * (1.0 / np.sqrt(float(DH))) + mask
        m = jnp.max(sc, axis=-1, keepdims=True)
        e = jnp.exp(sc - m)
        p = e / jnp.sum(e, axis=-1, keepdims=True)
        pieces.append(_dot(p, v))
    o = jnp.concatenate(pieces, axis=1)
    out_ref[0] = hs + _dot(o, wo_ref[...])


def _router_kernel(h_ref, s_ref, b_ref, wr_ref, su_ref, sl_ref,
                   y2_ref, pg_ref, ps_ref, te_ref, cmb_ref):
    h = h_ref[...]
    y2 = _ln_in(h, s_ref[...], b_ref[...])
    y2_ref[...] = y2
    logits = _dot(y2, wr_ref[...])
    col = lax.broadcasted_iota(jnp.int32, (NTOK, 128), 1)
    logits = jnp.where(col < NEXP, logits, -1e30)
    mx = jnp.max(logits, axis=-1, keepdims=True)
    ex = jnp.exp(logits - mx)
    probs = ex / jnp.sum(ex, axis=-1, keepdims=True)
    su = su_ref[...]
    m1 = jnp.max(probs, axis=-1, keepdims=True)
    eq1 = (probs == m1).astype(jnp.float32)
    first = eq1 * (1.0 - jnp.minimum(_dot(eq1, su), 1.0))
    probs2 = probs - first * 2.0
    m2 = jnp.max(probs2, axis=-1, keepdims=True)
    eq2 = (probs2 == m2).astype(jnp.float32)
    second = eq2 * (1.0 - jnp.minimum(_dot(eq2, su), 1.0))
    denom = m1 + m2
    w1 = m1 / denom
    w2 = m2 / denom
    row = lax.broadcasted_iota(jnp.int32, (NTOK, 128), 0)
    validf = ((row % LP) < L).astype(jnp.float32)
    first = first * validf
    second = second * validf
    sl = sl_ref[...]
    cnt1 = jnp.sum(first, axis=0, keepdims=True)
    cnt2 = jnp.sum(second, axis=0, keepdims=True)
    r0 = _dot(sl, first)
    r1 = cnt1 + _dot(sl, second)
    cnt = cnt1 + cnt2
    pc = jnp.floor((cnt + float(TT - 1)) * (1.0 / TT)) * float(TT)
    base = _dot(pc, su)
    pos0 = jnp.sum(first * (base + r0), axis=-1, keepdims=True)
    pos1 = jnp.sum(second * (base + r1), axis=-1, keepdims=True)
    rowv1 = (lax.broadcasted_iota(jnp.int32, (NTOK, 1), 0) % LP) < L
    big = float(SLOTS + 7)
    pos0 = jnp.where(rowv1, pos0, big)
    pos1 = jnp.where(rowv1, pos1, big)
    gi = lax.broadcasted_iota(jnp.int32, (NTOK, SLOTS), 1)
    hit0 = (gi == pos0.astype(jnp.int32)).astype(jnp.float32)
    hit1 = (gi == pos1.astype(jnp.int32)).astype(jnp.float32)
    pg_ref[...] = hit0 + hit1
    ps_ref[...] = hit0 * w1 + hit1 * w2
    ebase = base * (1.0 / TT)
    etiles = pc * (1.0 / TT)
    eye = (lax.broadcasted_iota(jnp.int32, (128, 128), 0) ==
           lax.broadcasted_iota(jnp.int32, (128, 128), 1)).astype(jnp.float32)
    bcolT = _dot_t(eye, ebase)
    ncolT = _dot_t(eye, etiles)
    ti = lax.broadcasted_iota(jnp.int32, (1, MAXT), 1).astype(jnp.float32)
    ind = jnp.logical_and(ti >= bcolT, ti < bcolT + ncolT).astype(jnp.float32)
    evec = lax.broadcasted_iota(jnp.int32, (1, 128), 1).astype(jnp.float32)
    te = _dot(evec, ind)
    te_ref[...] = jnp.broadcast_to(te, (8, MAXT)).astype(jnp.int32)
    cmb_ref[...] = first * w1 + second * w2


def _moe_grouped_kernel(te_ref, pg_ref, ps_ref, y2_ref, h_ref,
                        we1_ref, we2_ref, out_ref):
    t = pl.program_id(0)

    @pl.when(t == 0)
    def _():
        out_ref[...] = h_ref[...]

    x = _dot_tl(pg_ref[...], y2_ref[...])
    a = jax.nn.gelu(_dot(x.astype(jnp.bfloat16), we1_ref[0]))
    o = _dot(a.astype(jnp.bfloat16), we2_ref[0])
    out_ref[...] += _dot(ps_ref[...], o)


def _moe_dense_kernel(y2_ref, cmb_ref, h_ref, we1_ref, we2_ref, out_ref):
    e = pl.program_id(0)

    @pl.when(e == 0)
    def _():
        out_ref[...] = h_ref[...]

    bf16 = jnp.bfloat16
    a = jax.nn.gelu(_dot(y2_ref[...].astype(bf16), we1_ref[0].astype(bf16)))
    o = _dot(a.astype(bf16), we2_ref[0].astype(bf16))
    col = lax.broadcasted_iota(jnp.int32, (NTOK, 128), 1)
    w = jnp.sum(jnp.where(col == e, cmb_ref[...], 0.0), axis=-1, keepdims=True)
    out_ref[...] += o * w


def _head_kernel(clst_ref, wcls_ref, bcls_ref, cat_ref, m_ref, out_ref):
    proj = _dot(clst_ref[...], wcls_ref[...]) + bcls_ref[...]
    pn = proj / (jnp.sqrt(jnp.sum(proj * proj, axis=-1, keepdims=True)) + 1e-8)
    ct = cat_ref[...]
    cn = ct / (jnp.sqrt(jnp.sum(ct * ct, axis=-1, keepdims=True)) + 1e-8)
    sim = _dot_t(pn, cn)
    out_ref[...] = _dot(m_ref[...], sim)


@jax.jit
def _run(x_enc, W_patch, W_seq, pos_emb, ln1_s, ln1_b, Wqkv, Wo,
         ln2_s, ln2_b, Wr, We1, We2, Wcls, bcls, cat_tok):
    f32 = jnp.float32
    xt = jnp.transpose(x_enc, (0, 2, 1)).reshape(NSEQ, T)
    patches = xt.reshape(NSEQ * NPATCH, P)
    pos_p = jnp.zeros((LP, C), f32).at[:L].set(pos_emb)

    h = pl.pallas_call(
        _tokenize_kernel,
        out_shape=jax.ShapeDtypeStruct((NTOK, C), f32),
    )(patches, xt, jnp.asarray(_DPC), jnp.asarray(_DPS), W_patch,
      jnp.asarray(_DSC), jnp.asarray(_DSS), W_seq, pos_p)

    su = jnp.asarray(_SU128)
    sl = jnp.asarray(_SL640)
    wr_p = jnp.zeros((C, 128), f32)
    full = lambda shp: pl.BlockSpec(shp, lambda *_: (0,) * len(shp))

    for l in range(LAYERS):
        h3 = h.reshape(NSEQ, LP, C)
        h3 = pl.pallas_call(
            _attn_kernel,
            grid=(NSEQ,),
            in_specs=[
                pl.BlockSpec((1, LP, C), lambda s: (s, 0, 0)),
                full((1, C)), full((1, C)),
                full((C, 3 * C)), full((C, C)),
            ],
            out_specs=pl.BlockSpec((1, LP, C), lambda s: (s, 0, 0)),
            out_shape=jax.ShapeDtypeStruct((NSEQ, LP, C), f32),
        )(h3, ln1_s[l][None], ln1_b[l][None], Wqkv[l], Wo[l])
        h = h3.reshape(NTOK, C)

        y2, pg, ps, te, cmb = pl.pallas_call(
            _router_kernel,
            out_shape=[jax.ShapeDtypeStruct((NTOK, C), f32),
                       jax.ShapeDtypeStruct((NTOK, SLOTS), f32),
                       jax.ShapeDtypeStruct((NTOK, SLOTS), f32),
                       jax.ShapeDtypeStruct((8, MAXT), jnp.int32),
                       jax.ShapeDtypeStruct((NTOK, 128), f32)],
        )(h, ln2_s[l][None], ln2_b[l][None], wr_p.at[:, :NEXP].set(Wr[l]),
          su, sl)

        h = pl.pallas_call(
            _moe_dense_kernel,
            grid=(NEXP,),
            in_specs=[
                full((NTOK, C)), full((NTOK, 128)), full((NTOK, C)),
                pl.BlockSpec((1, C, DFF), lambda e: (e, 0, 0)),
                pl.BlockSpec((1, DFF, C), lambda e: (e, 0, 0)),
            ],
            out_specs=full((NTOK, C)),
            out_shape=jax.ShapeDtypeStruct((NTOK, C), f32),
        )(y2, cmb, h, We1[l], We2[l])

    clst = h.reshape(NSEQ, LP, C)[:, 0, :]
    cat_p = jnp.zeros((16, C), f32).at[:K].set(cat_tok)
    mmat = np.zeros((8, 16), np.float32)
    for b in range(B):
        mmat[b, b * V:(b + 1) * V] = 1.0 / V
    out = pl.pallas_call(
        _head_kernel,
        out_shape=jax.ShapeDtypeStruct((8, 16), f32),
    )(clst, Wcls, bcls[None], cat_p, jnp.asarray(mmat))
    return out[:B, :K]


def kernel(x_enc, x_mark_enc, W_patch, W_seq, pos_emb, ln1_s, ln1_b, Wqkv, Wo,
           ln2_s, ln2_b, Wr, We1, We2, Wcls, bcls, cat_tok):
    return _run(x_enc, W_patch, W_seq, pos_emb, ln1_s, ln1_b, Wqkv, Wo,
                ln2_s, ln2_b, Wr, We1, We2, Wcls, bcls, cat_tok)


# speedup vs baseline: 2.9054x; 2.6932x over previous
import numpy as np
import jax
import jax.numpy as jnp
from jax import lax
from jax.experimental import pallas as pl
from jax.experimental.pallas import tpu as pltpu

B = 2; T = 2048; V = 8; P = 64; NPATCH = T // P; C = 768; H = 12; DH = C // H
LAYERS = 2; NEXP = 8; TOPK = 2; DFF = 3072; K = 10; L = NPATCH + 1
FREQ_P = P // 2 + 1; FREQ_S = T // 2 + 1
NSEQ = B * V
LP = 40
NTOK = NSEQ * LP
NCH = 2
HC = DFF // NCH
NBUF = 2
NUNIT = LAYERS * NEXP * NCH


def _dft_mats(n, nfreq):
    ns = np.arange(n)[:, None]
    ks = np.arange(nfreq)[None, :]
    ang = 2.0 * np.pi * ns * ks / n
    return np.cos(ang).astype(np.float32), np.sin(ang).astype(np.float32)

_DPC, _DPS = _dft_mats(P, FREQ_P)
_DSC, _DSS = _dft_mats(T, FREQ_S)
_SU128 = np.triu(np.ones((128, 128), np.float32), 1)


def _ln_in(x, s, b):
    m = jnp.mean(x, axis=-1, keepdims=True)
    v = jnp.mean((x - m) * (x - m), axis=-1, keepdims=True)
    return (x - m) * lax.rsqrt(v + 1e-6) * s + b


def _dot(a, b):
    return jnp.dot(a, b, preferred_element_type=jnp.float32)


def _dot_t(a, b):
    return lax.dot_general(a, b, (((1,), (1,)), ((), ())),
                           preferred_element_type=jnp.float32)


def _unit(u):
    l, r = divmod(u, NEXP * NCH)
    e, c = divmod(r, NCH)
    return l, e, c


def _mega_kernel(patches_ref, x_ref, dpc_ref, dps_ref, wp_ref,
                 dsc_ref, dss_ref, ws_ref, pos_ref,
                 ln1s_ref, ln1b_ref, ln2s_ref, ln2b_ref,
                 wqkv_ref, wo_ref, wr_ref, su_ref,
                 wcls_ref, bcls_ref, cat_ref, m_ref,
                 we1_hbm, we2_hbm,
                 out_ref,
                 h_s, y2b_s, cmb_s, o_s, w1buf, w2buf, sem1, sem2):
    bf16 = jnp.bfloat16

    def _issue(u):
        l, e, c = _unit(u)
        s = u % NBUF
        pltpu.make_async_copy(
            we1_hbm.at[l, e, :, pl.ds(c * HC, HC)], w1buf.at[s], sem1.at[s]
        ).start()
        pltpu.make_async_copy(
            we2_hbm.at[l, e, pl.ds(c * HC, HC), :], w2buf.at[s], sem2.at[s]
        ).start()

    def _wait(u):
        l, e, c = _unit(u)
        s = u % NBUF
        pltpu.make_async_copy(
            we1_hbm.at[l, e, :, pl.ds(c * HC, HC)], w1buf.at[s], sem1.at[s]
        ).wait()
        pltpu.make_async_copy(
            we2_hbm.at[l, e, pl.ds(c * HC, HC), :], w2buf.at[s], sem2.at[s]
        ).wait()

    for u in range(NBUF):
        _issue(u)

    pr = patches_ref[...]
    re = _dot(pr, dpc_ref[...])
    im = _dot(pr, dps_ref[...])
    pf = jnp.sqrt(re * re + im * im)
    tok = _dot(pf, wp_ref[...])
    xr = x_ref[...]
    sre = _dot(xr.astype(bf16), dsc_ref[...])
    sim_ = _dot(xr.astype(bf16), dss_ref[...])
    sf = jnp.sqrt(sre * sre + sim_ * sim_)
    cls = _dot(sf, ws_ref[...])
    pos = pos_ref[...]
    zero = jnp.zeros((LP - L, C), jnp.float32)
    for s in range(NSEQ):
        h_s[s * LP:s * LP + 1, :] = cls[s:s + 1, :] + pos[0:1, :]
        h_s[s * LP + 1:s * LP + L, :] = tok[s * NPATCH:(s + 1) * NPATCH, :] + pos[1:L, :]
        h_s[s * LP + L:(s + 1) * LP, :] = zero

    col_mask = lax.broadcasted_iota(jnp.int32, (LP, LP), 1)
    amask = jnp.where(col_mask < L, 0.0, -1e9).astype(jnp.float32)
    ecol = lax.broadcasted_iota(jnp.int32, (NTOK, 128), 1)
    rowv = (lax.broadcasted_iota(jnp.int32, (NTOK, 128), 0) % LP) < L

    unit_base = 0
    for l in range(LAYERS):
        hv = h_s[...]
        y = _ln_in(hv, ln1s_ref[l:l + 1, :], ln1b_ref[l:l + 1, :])
        qkv = _dot(y.astype(bf16), wqkv_ref[l])
        for s in range(NSEQ):
            r0 = s * LP
            qs = qkv[r0:r0 + LP, :]
            for hh in range(H):
                q = qs[:, hh * DH:(hh + 1) * DH]
                k = qs[:, C + hh * DH:C + (hh + 1) * DH]
                v = qs[:, 2 * C + hh * DH:2 * C + (hh + 1) * DH]
                sc = _dot_t(q, k) * (1.0 / np.sqrt(float(DH))) + amask
                mx = jnp.max(sc, axis=-1, keepdims=True)
                ex = jnp.exp(sc - mx)
                p = ex / jnp.sum(ex, axis=-1, keepdims=True)
                o_s[r0:r0 + LP, hh * DH:(hh + 1) * DH] = _dot(p, v)
        h_s[...] = hv + _dot(o_s[...].astype(bf16), wo_ref[l])

        hv = h_s[...]
        y2 = _ln_in(hv, ln2s_ref[l:l + 1, :], ln2b_ref[l:l + 1, :])
        y2b_s[...] = y2.astype(bf16)
        logits = _dot(y2, wr_ref[l])
        logits = jnp.where(ecol < NEXP, logits, -1e30)
        mx = jnp.max(logits, axis=-1, keepdims=True)
        ex = jnp.exp(logits - mx)
        probs = ex / jnp.sum(ex, axis=-1, keepdims=True)
        su = su_ref[...]
        m1 = jnp.max(probs, axis=-1, keepdims=True)
        eq1 = (probs == m1).astype(jnp.float32)
        first = eq1 * (1.0 - jnp.minimum(_dot(eq1, su), 1.0))
        probs2 = probs - first * 2.0
        m2 = jnp.max(probs2, axis=-1, keepdims=True)
        eq2 = (probs2 == m2).astype(jnp.float32)
        second = eq2 * (1.0 - jnp.minimum(_dot(eq2, su), 1.0))
        denom = m1 + m2
        cmb = first * (m1 / denom) + second * (m2 / denom)
        cmb_s[...] = jnp.where(rowv, cmb, 0.0)

        for r in range(NEXP * NCH):
            u = unit_base + r
            _, e, c = _unit(u)
            _wait(u)
            su_ = u % NBUF
            w1c = w1buf[su_].astype(bf16)
            w2c = w2buf[su_].astype(bf16)
            a = jax.nn.gelu(_dot(y2b_s[...], w1c))
            o = _dot(a.astype(bf16), w2c)
            if u + NBUF < NUNIT:
                _issue(u + NBUF)
            h_s[...] += o * cmb_s[:, e:e + 1]
        unit_base += NEXP * NCH

    clst = jnp.concatenate([h_s[s * LP:s * LP + 1, :] for s in range(NSEQ)],
                           axis=0)
    proj = _dot(clst, wcls_ref[...]) + bcls_ref[...]
    pn = proj / (jnp.sqrt(jnp.sum(proj * proj, axis=-1, keepdims=True)) + 1e-8)
    ct = cat_ref[...]
    cn = ct / (jnp.sqrt(jnp.sum(ct * ct, axis=-1, keepdims=True)) + 1e-8)
    sim = _dot_t(pn, cn)
    out_ref[...] = _dot(m_ref[...], sim)


@jax.jit
def _run(x_enc, W_patch, W_seq, pos_emb, ln1_s, ln1_b, Wqkv, Wo,
         ln2_s, ln2_b, Wr, We1, We2, Wcls, bcls, cat_tok):
    f32 = jnp.float32
    bf16 = jnp.bfloat16
    xt = jnp.transpose(x_enc, (0, 2, 1)).reshape(NSEQ, T)
    patches = xt.reshape(NSEQ * NPATCH, P)
    pos_p = jnp.zeros((LP, C), f32).at[:L].set(pos_emb)
    wr_p = jnp.zeros((LAYERS, C, 128), f32).at[:, :, :NEXP].set(Wr)
    cat_p = jnp.zeros((16, C), f32).at[:K].set(cat_tok)
    mmat = np.zeros((8, 16), np.float32)
    for b in range(B):
        mmat[b, b * V:(b + 1) * V] = 1.0 / V

    vspec = pl.BlockSpec(memory_space=pltpu.VMEM)
    aspec = pl.BlockSpec(memory_space=pltpu.MemorySpace.HBM)
    out = pl.pallas_call(
        _mega_kernel,
        in_specs=[vspec] * 21 + [aspec, aspec],
        out_specs=vspec,
        out_shape=jax.ShapeDtypeStruct((8, 16), f32),
        scratch_shapes=[
            pltpu.VMEM((NTOK, C), f32),
            pltpu.VMEM((NTOK, C), bf16),
            pltpu.VMEM((NTOK, 128), f32),
            pltpu.VMEM((NTOK, C), f32),
            pltpu.VMEM((NBUF, C, HC), f32),
            pltpu.VMEM((NBUF, HC, C), f32),
            pltpu.SemaphoreType.DMA((NBUF,)),
            pltpu.SemaphoreType.DMA((NBUF,)),
        ],
        compiler_params=pltpu.CompilerParams(
            vmem_limit_bytes=120 * 1024 * 1024,
        ),
    )(patches, xt, jnp.asarray(_DPC), jnp.asarray(_DPS), W_patch,
      jnp.asarray(_DSC).astype(bf16), jnp.asarray(_DSS).astype(bf16), W_seq,
      pos_p, ln1_s, ln1_b, ln2_s, ln2_b, Wqkv.astype(bf16), Wo.astype(bf16), wr_p,
      jnp.asarray(_SU128), Wcls, bcls[None], cat_p, jnp.asarray(mmat),
      We1, We2)
    return out[:B, :K]


def kernel(x_enc, x_mark_enc, W_patch, W_seq, pos_emb, ln1_s, ln1_b, Wqkv, Wo,
           ln2_s, ln2_b, Wr, We1, We2, Wcls, bcls, cat_tok):
    return _run(x_enc, W_patch, W_seq, pos_emb, ln1_s, ln1_b, Wqkv, Wo,
                ln2_s, ln2_b, Wr, We1, We2, Wcls, bcls, cat_tok)
